# Initial kernel scaffold; baseline (speedup 1.0000x reference)
import jax
import jax.numpy as jnp
from jax.experimental import pallas as pl


def kernel(node_feats, batch, W1, b1, W2, b2):
    raise NotImplementedError("write your pallas kernel here")


# speedup vs baseline: 2.5562x; 2.5562x over previous
import functools

import jax
import jax.numpy as jnp
from jax import lax
from jax.experimental import pallas as pl
from jax.experimental.pallas import tpu as pltpu
from jax.experimental.pallas import tpu_sc as plsc

N_NODES = 100000
D_FEAT = 128
HIDDEN = 64
NUM_SEGMENTS = 512

ROWS = 2048
N_PAD = 100352
GRID = N_PAD // ROWS
NW = 16
CHUNK_ROWS = 49
ACC = 640


def _mlp_body(x_ref, w1_ref, b1_ref, w2r_ref, b2_ref, o_ref):
    x = x_ref[...]
    h = lax.dot_general(x, w1_ref[...], (((1,), (0,)), ((), ())),
                        preferred_element_type=jnp.float32)
    h = h + b1_ref[...]
    h = h * (1.0 / (1.0 + jnp.exp(-h)))
    c = jnp.sum(h * w2r_ref[...], axis=1)
    o_ref[...] = (c + b2_ref[0, 0]).reshape(1, 1, ROWS)


def _mlp_contrib(x, w1, b1r, w2r, b2r):
    return pl.pallas_call(
        _mlp_body,
        grid=(GRID,),
        in_specs=[
            pl.BlockSpec((ROWS, D_FEAT), lambda i: (i, 0)),
            pl.BlockSpec((D_FEAT, HIDDEN), lambda i: (0, 0)),
            pl.BlockSpec((1, HIDDEN), lambda i: (0, 0)),
            pl.BlockSpec((1, HIDDEN), lambda i: (0, 0)),
            pl.BlockSpec((1, 1), lambda i: (0, 0)),
        ],
        out_specs=pl.BlockSpec((1, 1, ROWS), lambda i: (i, 0, 0)),
        out_shape=jax.ShapeDtypeStruct((GRID, 1, ROWS), jnp.float32),
    )(x, w1, b1r, w2r, b2r)


def _seg_body(contrib_hbm, batch_hbm, zeros_hbm, ones_hbm, out_hbm,
              vals_v, idx_v, ones_v, ssum_st, scnt_st, out_st, ssum, scnt):
    wid = lax.axis_index("s")

    @pl.when(wid == 0)
    def _init():
        pltpu.sync_copy(zeros_hbm, ssum)
        pltpu.sync_copy(zeros_hbm, scnt)

    pltpu.sync_copy(contrib_hbm.at[wid], vals_v)
    pltpu.sync_copy(batch_hbm.at[wid], idx_v)
    pltpu.sync_copy(ones_hbm, ones_v)
    plsc.subcore_barrier()

    def body(j, carry):
        pltpu.sync_copy(vals_v.at[j], ssum.at[idx_v.at[j]], add=True)
        pltpu.sync_copy(ones_v, scnt.at[idx_v.at[j]], add=True)
        return carry

    lax.fori_loop(0, CHUNK_ROWS, body, 0)
    plsc.subcore_barrier()

    @pl.when(wid == 0)
    def _finish():
        pltpu.sync_copy(ssum, ssum_st)
        pltpu.sync_copy(scnt, scnt_st)
        for k in range(NUM_SEGMENTS // 16):
            s = ssum_st[pl.ds(k * 16, 16)]
            c = scnt_st[pl.ds(k * 16, 16)]
            out_st[pl.ds(k * 16, 16)] = s / jnp.maximum(c, 1.0)
        pltpu.sync_copy(out_st, out_hbm)


@functools.cache
def _seg_mean():
    return pl.kernel(
        _seg_body,
        out_type=jax.ShapeDtypeStruct((NUM_SEGMENTS,), jnp.float32),
        mesh=plsc.VectorSubcoreMesh(core_axis_name="c", subcore_axis_name="s",
                                    num_cores=1, num_subcores=16),
        scratch_types=[
            pltpu.VMEM((CHUNK_ROWS, 128), jnp.float32),
            pltpu.VMEM((CHUNK_ROWS, 128), jnp.int32),
            pltpu.VMEM((128,), jnp.float32),
            pltpu.VMEM((ACC,), jnp.float32),
            pltpu.VMEM((ACC,), jnp.float32),
            pltpu.VMEM((NUM_SEGMENTS,), jnp.float32),
            pltpu.VMEM_SHARED((ACC,), jnp.float32),
            pltpu.VMEM_SHARED((ACC,), jnp.float32),
        ],
    )


def kernel(node_feats, batch, W1, b1, W2, b2):
    b1r = b1.reshape(1, HIDDEN)
    w2r = W2.reshape(1, HIDDEN)
    b2r = b2.reshape(1, 1)
    contrib = _mlp_contrib(node_feats, W1, b1r, w2r, b2r)
    contrib = contrib.reshape(NW, CHUNK_ROWS, 128)

    batch_i = jnp.pad(batch.astype(jnp.int32), (0, N_PAD - N_NODES),
                      constant_values=NUM_SEGMENTS)
    batch_i = batch_i.reshape(NW, CHUNK_ROWS, 128)

    zeros = jnp.zeros((ACC,), jnp.float32)
    ones = jnp.ones((128,), jnp.float32)
    return _seg_mean()(contrib, batch_i, zeros, ones)


# speedup vs baseline: 3.2210x; 1.2601x over previous
import functools

import jax
import jax.numpy as jnp
from jax import lax
from jax.experimental import pallas as pl
from jax.experimental.pallas import tpu as pltpu
from jax.experimental.pallas import tpu_sc as plsc

N_NODES = 100000
D_FEAT = 128
HIDDEN = 64
NUM_SEGMENTS = 512

ROWS = 2048
N_PAD = 100352
GRID = N_PAD // ROWS
NW = 16
CHUNK_ROWS = 49
GROUP = 7
ACC = 640


def _mlp_body(x_ref, w1_ref, b1_ref, w2r_ref, b2_ref, o_ref):
    x = x_ref[...]
    h = lax.dot_general(x, w1_ref[...], (((1,), (0,)), ((), ())),
                        preferred_element_type=jnp.float32)
    h = h + b1_ref[...]
    h = h * (1.0 / (1.0 + jnp.exp(-h)))
    c = lax.dot_general(w2r_ref[...], h, (((1,), (1,)), ((), ())),
                        preferred_element_type=jnp.float32)
    o_ref[...] = (c + b2_ref[0, 0]).reshape(1, 1, ROWS)


def _mlp_contrib(x, w1, b1r, w2r, b2r):
    return pl.pallas_call(
        _mlp_body,
        grid=(GRID,),
        in_specs=[
            pl.BlockSpec((ROWS, D_FEAT), lambda i: (i, 0)),
            pl.BlockSpec((D_FEAT, HIDDEN), lambda i: (0, 0)),
            pl.BlockSpec((1, HIDDEN), lambda i: (0, 0)),
            pl.BlockSpec((1, HIDDEN), lambda i: (0, 0)),
            pl.BlockSpec((1, 1), lambda i: (0, 0)),
        ],
        out_specs=pl.BlockSpec((1, 1, ROWS), lambda i: (i, 0, 0)),
        out_shape=jax.ShapeDtypeStruct((GRID, 1, ROWS), jnp.float32),
    )(x, w1, b1r, w2r, b2r)


def _seg_body(contrib_hbm, batch_hbm, zeros_hbm, ones_hbm, out_hbm,
              vals_v, idx_v, ones_v, ssum_st, scnt_st, out_st, ssum, scnt,
              sem):
    wid = lax.axis_index("s")

    @pl.when(wid == 0)
    def _init():
        pltpu.sync_copy(zeros_hbm, ssum)
        pltpu.sync_copy(zeros_hbm, scnt)

    pltpu.sync_copy(contrib_hbm.at[wid], vals_v)
    pltpu.sync_copy(batch_hbm.at[wid], idx_v)
    pltpu.sync_copy(ones_hbm, ones_v)
    plsc.subcore_barrier()

    def body(g, carry):
        handles = []
        for u in range(GROUP):
            j = g * GROUP + u
            handles.append(pltpu.async_copy(
                vals_v.at[j], ssum.at[idx_v.at[j]], sem, add=True))
            handles.append(pltpu.async_copy(
                ones_v, scnt.at[idx_v.at[j]], sem, add=True))
        for h in handles:
            h.wait()
        return carry

    lax.fori_loop(0, CHUNK_ROWS // GROUP, body, 0)
    plsc.subcore_barrier()

    @pl.when(wid == 0)
    def _finish():
        pltpu.sync_copy(ssum, ssum_st)
        pltpu.sync_copy(scnt, scnt_st)
        for k in range(NUM_SEGMENTS // 16):
            s = ssum_st[pl.ds(k * 16, 16)]
            c = scnt_st[pl.ds(k * 16, 16)]
            out_st[pl.ds(k * 16, 16)] = s / jnp.maximum(c, 1.0)
        pltpu.sync_copy(out_st, out_hbm)


@functools.cache
def _seg_mean():
    return pl.kernel(
        _seg_body,
        out_type=jax.ShapeDtypeStruct((NUM_SEGMENTS,), jnp.float32),
        mesh=plsc.VectorSubcoreMesh(core_axis_name="c", subcore_axis_name="s",
                                    num_cores=1, num_subcores=16),
        scratch_types=[
            pltpu.VMEM((CHUNK_ROWS, 128), jnp.float32),
            pltpu.VMEM((CHUNK_ROWS, 128), jnp.int32),
            pltpu.VMEM((128,), jnp.float32),
            pltpu.VMEM((ACC,), jnp.float32),
            pltpu.VMEM((ACC,), jnp.float32),
            pltpu.VMEM((NUM_SEGMENTS,), jnp.float32),
            pltpu.VMEM_SHARED((ACC,), jnp.float32),
            pltpu.VMEM_SHARED((ACC,), jnp.float32),
            pltpu.SemaphoreType.DMA,
        ],
    )


def kernel(node_feats, batch, W1, b1, W2, b2):
    b1r = b1.reshape(1, HIDDEN)
    w2r = W2.reshape(1, HIDDEN)
    b2r = b2.reshape(1, 1)
    contrib = _mlp_contrib(node_feats, W1, b1r, w2r, b2r)
    contrib = contrib.reshape(NW, CHUNK_ROWS, 128)

    batch_i = jnp.pad(batch.astype(jnp.int32), (0, N_PAD - N_NODES),
                      constant_values=NUM_SEGMENTS)
    batch_i = batch_i.reshape(NW, CHUNK_ROWS, 128)

    zeros = jnp.zeros((ACC,), jnp.float32)
    ones = jnp.ones((128,), jnp.float32)
    return _seg_mean()(contrib, batch_i, zeros, ones)


# speedup vs baseline: 3.9602x; 1.2295x over previous
import functools

import jax
import jax.numpy as jnp
from jax import lax
from jax.experimental import pallas as pl
from jax.experimental.pallas import tpu as pltpu
from jax.experimental.pallas import tpu_sc as plsc

N_NODES = 100000
D_FEAT = 128
HIDDEN = 64
NUM_SEGMENTS = 512

ROWS = 4096
N_PAD = 102400
GRID = N_PAD // ROWS
NW = 16
CHUNK_ROWS = 50
GROUP = 10
ACC = 640


def _mlp_body(x_ref, w1_ref, b1_ref, w2r_ref, b2_ref, o_ref):
    x = x_ref[...]
    h = lax.dot_general(x, w1_ref[...], (((1,), (0,)), ((), ())),
                        preferred_element_type=jnp.float32)
    h = h + b1_ref[...]
    h = h * (1.0 / (1.0 + jnp.exp(-h)))
    c = lax.dot_general(w2r_ref[...], h, (((1,), (1,)), ((), ())),
                        preferred_element_type=jnp.float32)
    o_ref[...] = (c + b2_ref[0, 0]).reshape(1, 1, ROWS)


def _mlp_contrib(x, w1, b1r, w2r, b2r):
    return pl.pallas_call(
        _mlp_body,
        grid=(GRID,),
        in_specs=[
            pl.BlockSpec((ROWS, D_FEAT), lambda i: (i, 0)),
            pl.BlockSpec((D_FEAT, HIDDEN), lambda i: (0, 0)),
            pl.BlockSpec((1, HIDDEN), lambda i: (0, 0)),
            pl.BlockSpec((1, HIDDEN), lambda i: (0, 0)),
            pl.BlockSpec((1, 1), lambda i: (0, 0)),
        ],
        out_specs=pl.BlockSpec((1, 1, ROWS), lambda i: (i, 0, 0)),
        out_shape=jax.ShapeDtypeStruct((GRID, 1, ROWS), jnp.float32),
    )(x, w1, b1r, w2r, b2r)


def _seg_body(contrib_hbm, batch_hbm, zeros_hbm, ones_hbm, out_hbm,
              vals_v, idx_v, ones_v, ssum_st, scnt_st, out_st, ssum, scnt,
              sem):
    wid = lax.axis_index("s")

    @pl.when(wid == 0)
    def _init():
        pltpu.sync_copy(zeros_hbm, ssum)
        pltpu.sync_copy(zeros_hbm, scnt)

    pltpu.sync_copy(contrib_hbm.at[wid], vals_v)
    pltpu.sync_copy(batch_hbm.at[wid], idx_v)
    pltpu.sync_copy(ones_hbm, ones_v)
    plsc.subcore_barrier()

    def body(g, carry):
        handles = []
        for u in range(GROUP):
            j = g * GROUP + u
            handles.append(pltpu.async_copy(
                vals_v.at[j], ssum.at[idx_v.at[j]], sem, add=True))
            handles.append(pltpu.async_copy(
                ones_v, scnt.at[idx_v.at[j]], sem, add=True))
        for h in handles:
            h.wait()
        return carry

    lax.fori_loop(0, CHUNK_ROWS // GROUP, body, 0)
    plsc.subcore_barrier()

    @pl.when(wid == 0)
    def _finish():
        pltpu.sync_copy(ssum, ssum_st)
        pltpu.sync_copy(scnt, scnt_st)
        for k in range(NUM_SEGMENTS // 16):
            s = ssum_st[pl.ds(k * 16, 16)]
            c = scnt_st[pl.ds(k * 16, 16)]
            out_st[pl.ds(k * 16, 16)] = s / jnp.maximum(c, 1.0)
        pltpu.sync_copy(out_st, out_hbm)


@functools.cache
def _seg_mean():
    return pl.kernel(
        _seg_body,
        out_type=jax.ShapeDtypeStruct((NUM_SEGMENTS,), jnp.float32),
        mesh=plsc.VectorSubcoreMesh(core_axis_name="c", subcore_axis_name="s",
                                    num_cores=1, num_subcores=16),
        scratch_types=[
            pltpu.VMEM((CHUNK_ROWS, 128), jnp.float32),
            pltpu.VMEM((CHUNK_ROWS, 128), jnp.int32),
            pltpu.VMEM((128,), jnp.float32),
            pltpu.VMEM((ACC,), jnp.float32),
            pltpu.VMEM((ACC,), jnp.float32),
            pltpu.VMEM((NUM_SEGMENTS,), jnp.float32),
            pltpu.VMEM_SHARED((ACC,), jnp.float32),
            pltpu.VMEM_SHARED((ACC,), jnp.float32),
            pltpu.SemaphoreType.DMA,
        ],
    )


def kernel(node_feats, batch, W1, b1, W2, b2):
    b1r = b1.reshape(1, HIDDEN)
    w2r = W2.reshape(1, HIDDEN)
    b2r = b2.reshape(1, 1)
    contrib = _mlp_contrib(node_feats, W1, b1r, w2r, b2r)
    contrib = contrib.reshape(NW, CHUNK_ROWS, 128)

    batch_i = jnp.pad(batch.astype(jnp.int32), (0, N_PAD - N_NODES),
                      constant_values=NUM_SEGMENTS)
    batch_i = batch_i.reshape(NW, CHUNK_ROWS, 128)

    zeros = jnp.zeros((ACC,), jnp.float32)
    ones = jnp.ones((128,), jnp.float32)
    return _seg_mean()(contrib, batch_i, zeros, ones)


# speedup vs baseline: 4.2034x; 1.0614x over previous
import functools

import jax
import jax.numpy as jnp
from jax import lax
from jax.experimental import pallas as pl
from jax.experimental.pallas import tpu as pltpu
from jax.experimental.pallas import tpu_sc as plsc

N_NODES = 100000
D_FEAT = 128
HIDDEN = 64
NUM_SEGMENTS = 512

ROWS = 4096
N_PAD = 102400
GRID = N_PAD // ROWS
NC = 2
NS = 16
NW = NC * NS
CHUNK_ROWS = N_PAD // (NW * 128)
GROUP = 5
ACC = 640


def _mlp_body(x_ref, w1_ref, b1_ref, w2r_ref, b2_ref, o_ref):
    x = x_ref[...]
    h = lax.dot_general(x, w1_ref[...], (((1,), (0,)), ((), ())),
                        preferred_element_type=jnp.float32)
    h = h + b1_ref[...]
    h = h * (1.0 / (1.0 + jnp.exp(-h)))
    c = lax.dot_general(w2r_ref[...], h, (((1,), (1,)), ((), ())),
                        preferred_element_type=jnp.float32)
    o_ref[...] = (c + b2_ref[0, 0]).reshape(ROWS)


def _mlp_contrib(x, w1, b1r, w2r, b2r):
    return pl.pallas_call(
        _mlp_body,
        grid=(GRID,),
        in_specs=[
            pl.BlockSpec((ROWS, D_FEAT), lambda i: (i, 0)),
            pl.BlockSpec((D_FEAT, HIDDEN), lambda i: (0, 0)),
            pl.BlockSpec((1, HIDDEN), lambda i: (0, 0)),
            pl.BlockSpec((1, HIDDEN), lambda i: (0, 0)),
            pl.BlockSpec((1, 1), lambda i: (0, 0)),
        ],
        out_specs=pl.BlockSpec((ROWS,), lambda i: (i,)),
        out_shape=jax.ShapeDtypeStruct((N_PAD,), jnp.float32),
    )(x, w1, b1r, w2r, b2r)


def _seg_body(contrib_hbm, batch_hbm, zeros_hbm, ones_hbm, out_hbm,
              vals_v, idx_v, ones_v, ssum, scnt, sem):
    cid = lax.axis_index("c")
    sid = lax.axis_index("s")
    wid = sid * NC + cid

    @pl.when(sid == 0)
    def _init():
        pltpu.sync_copy(zeros_hbm, ssum)
        pltpu.sync_copy(zeros_hbm, scnt)

    pltpu.sync_copy(contrib_hbm.at[wid], vals_v)
    pltpu.sync_copy(batch_hbm.at[wid], idx_v)
    pltpu.sync_copy(ones_hbm, ones_v)
    plsc.subcore_barrier()

    def body(g, carry):
        handles = []
        for u in range(GROUP):
            j = g * GROUP + u
            handles.append(pltpu.async_copy(
                vals_v.at[j], ssum.at[idx_v.at[j]], sem, add=True))
            handles.append(pltpu.async_copy(
                ones_v, scnt.at[idx_v.at[j]], sem, add=True))
        for h in handles:
            h.wait()
        return carry

    lax.fori_loop(0, CHUNK_ROWS // GROUP, body, 0)
    plsc.subcore_barrier()

    @pl.when(sid == 0)
    def _publish():
        pltpu.sync_copy(ssum, out_hbm.at[cid, 0])
        pltpu.sync_copy(scnt, out_hbm.at[cid, 1])


@functools.cache
def _seg_partials():
    return pl.kernel(
        _seg_body,
        out_type=jax.ShapeDtypeStruct((NC, 2, ACC), jnp.float32),
        mesh=plsc.VectorSubcoreMesh(core_axis_name="c", subcore_axis_name="s",
                                    num_cores=NC, num_subcores=NS),
        scratch_types=[
            pltpu.VMEM((CHUNK_ROWS, 128), jnp.float32),
            pltpu.VMEM((CHUNK_ROWS, 128), jnp.int32),
            pltpu.VMEM((128,), jnp.float32),
            pltpu.VMEM_SHARED((ACC,), jnp.float32),
            pltpu.VMEM_SHARED((ACC,), jnp.float32),
            pltpu.SemaphoreType.DMA,
        ],
    )


def _combine_body(p_ref, o_ref):
    p = p_ref[...]
    sums = p[0, 0, :NUM_SEGMENTS] + p[1, 0, :NUM_SEGMENTS]
    cnts = p[0, 1, :NUM_SEGMENTS] + p[1, 1, :NUM_SEGMENTS]
    o_ref[...] = (sums / jnp.maximum(cnts, 1.0)).reshape(1, NUM_SEGMENTS)


def _combine(partials):
    return pl.pallas_call(
        _combine_body,
        out_shape=jax.ShapeDtypeStruct((1, NUM_SEGMENTS), jnp.float32),
    )(partials)


def kernel(node_feats, batch, W1, b1, W2, b2):
    b1r = b1.reshape(1, HIDDEN)
    w2r = W2.reshape(1, HIDDEN)
    b2r = b2.reshape(1, 1)
    contrib = _mlp_contrib(node_feats, W1, b1r, w2r, b2r)
    contrib = contrib.reshape(NW, CHUNK_ROWS, 128)

    batch_i = jnp.pad(batch.astype(jnp.int32), (0, N_PAD - N_NODES),
                      constant_values=NUM_SEGMENTS)
    batch_i = batch_i.reshape(NW, CHUNK_ROWS, 128)

    zeros = jnp.zeros((ACC,), jnp.float32)
    ones = jnp.ones((128,), jnp.float32)
    partials = _seg_partials()(contrib, batch_i, zeros, ones)
    return _combine(partials).reshape(NUM_SEGMENTS)


# speedup vs baseline: 4.4891x; 1.0680x over previous
import functools

import jax
import jax.numpy as jnp
from jax import lax
from jax.experimental import pallas as pl
from jax.experimental.pallas import tpu as pltpu
from jax.experimental.pallas import tpu_sc as plsc

N_NODES = 100000
D_FEAT = 128
HIDDEN = 64
NUM_SEGMENTS = 512

ROWS = 4096
N_PAD = 102400
GRID = N_PAD // ROWS
NC = 2
NS = 16
NW = NC * NS
CHUNK_ROWS = N_PAD // (NW * 128)
CHUNK = CHUNK_ROWS * 128
GROUP = 5
ACC = 640


def _mlp_body(x_ref, w1_ref, b1_ref, w2r_ref, b2_ref, o_ref):
    x = x_ref[...]
    h = lax.dot_general(x, w1_ref[...], (((1,), (0,)), ((), ())),
                        preferred_element_type=jnp.float32)
    h = h + b1_ref[...]
    h = h * (1.0 / (1.0 + jnp.exp(-h)))
    c = lax.dot_general(w2r_ref[...], h, (((1,), (1,)), ((), ())),
                        preferred_element_type=jnp.float32)
    o_ref[...] = (c + b2_ref[0, 0]).reshape(ROWS)


def _mlp_contrib(x, w1, b1r, w2r, b2r):
    return pl.pallas_call(
        _mlp_body,
        grid=(GRID,),
        in_specs=[
            pl.BlockSpec((ROWS, D_FEAT), lambda i: (i, 0)),
            pl.BlockSpec((D_FEAT, HIDDEN), lambda i: (0, 0)),
            pl.BlockSpec((1, HIDDEN), lambda i: (0, 0)),
            pl.BlockSpec((1, HIDDEN), lambda i: (0, 0)),
            pl.BlockSpec((1, 1), lambda i: (0, 0)),
        ],
        out_specs=pl.BlockSpec((ROWS,), lambda i: (i,)),
        out_shape=jax.ShapeDtypeStruct((N_PAD,), jnp.float32),
    )(x, w1, b1r, w2r, b2r)


def _seg_body(contrib_hbm, batch_hbm, out_hbm,
              vals_v, idx_v, ones_v, zeros_v, ssum, scnt, sem):
    cid = lax.axis_index("c")
    sid = lax.axis_index("s")
    wid = sid * NC + cid

    one16 = jnp.ones((16,), jnp.float32)
    for k in range(8):
        ones_v[pl.ds(k * 16, 16)] = one16

    @pl.when(sid == 0)
    def _init():
        zero16 = jnp.zeros((16,), jnp.float32)
        for k in range(ACC // 16):
            zeros_v[pl.ds(k * 16, 16)] = zero16
        pltpu.sync_copy(zeros_v, ssum)
        pltpu.sync_copy(zeros_v, scnt)

    pltpu.sync_copy(contrib_hbm.at[pl.ds(wid * CHUNK, CHUNK)], vals_v)
    pltpu.sync_copy(batch_hbm.at[wid], idx_v)
    plsc.subcore_barrier()

    def body(g, carry):
        handles = []
        for u in range(GROUP):
            j = g * GROUP + u
            handles.append(pltpu.async_copy(
                vals_v.at[pl.ds(j * 128, 128)], ssum.at[idx_v.at[j]],
                sem, add=True))
            handles.append(pltpu.async_copy(
                ones_v, scnt.at[idx_v.at[j]], sem, add=True))
        for h in handles:
            h.wait()
        return carry

    lax.fori_loop(0, CHUNK_ROWS // GROUP, body, 0)
    plsc.subcore_barrier()

    @pl.when(sid == 0)
    def _publish():
        pltpu.sync_copy(ssum, out_hbm.at[cid, 0])
        pltpu.sync_copy(scnt, out_hbm.at[cid, 1])


@functools.cache
def _seg_partials():
    return pl.kernel(
        _seg_body,
        out_type=jax.ShapeDtypeStruct((NC, 2, ACC), jnp.float32),
        mesh=plsc.VectorSubcoreMesh(core_axis_name="c", subcore_axis_name="s",
                                    num_cores=NC, num_subcores=NS),
        scratch_types=[
            pltpu.VMEM((CHUNK,), jnp.float32),
            pltpu.VMEM((CHUNK_ROWS, 128), jnp.int32),
            pltpu.VMEM((128,), jnp.float32),
            pltpu.VMEM((ACC,), jnp.float32),
            pltpu.VMEM_SHARED((ACC,), jnp.float32),
            pltpu.VMEM_SHARED((ACC,), jnp.float32),
            pltpu.SemaphoreType.DMA,
        ],
    )


def _combine_body(p_ref, o_ref):
    p = p_ref[...]
    sums = p[0, 0, :NUM_SEGMENTS] + p[1, 0, :NUM_SEGMENTS]
    cnts = p[0, 1, :NUM_SEGMENTS] + p[1, 1, :NUM_SEGMENTS]
    o_ref[...] = (sums / jnp.maximum(cnts, 1.0)).reshape(1, NUM_SEGMENTS)


def _combine(partials):
    return pl.pallas_call(
        _combine_body,
        out_shape=jax.ShapeDtypeStruct((1, NUM_SEGMENTS), jnp.float32),
    )(partials)


def kernel(node_feats, batch, W1, b1, W2, b2):
    b1r = b1.reshape(1, HIDDEN)
    w2r = W2.reshape(1, HIDDEN)
    b2r = b2.reshape(1, 1)
    contrib = _mlp_contrib(node_feats, W1, b1r, w2r, b2r)

    batch_i = jnp.pad(batch.astype(jnp.int32), (0, N_PAD - N_NODES),
                      constant_values=NUM_SEGMENTS)
    batch_i = batch_i.reshape(NW, CHUNK_ROWS, 128)

    partials = _seg_partials()(contrib, batch_i)
    return _combine(partials).reshape(NUM_SEGMENTS)


# speedup vs baseline: 4.7034x; 1.0477x over previous
import functools

import jax
import jax.numpy as jnp
from jax import lax
from jax.experimental import pallas as pl
from jax.experimental.pallas import tpu as pltpu
from jax.experimental.pallas import tpu_sc as plsc

N_NODES = 100000
D_FEAT = 128
HIDDEN = 64
NUM_SEGMENTS = 512

ROWS = 5120
N_PAD = 102400
GRID = N_PAD // ROWS
NC = 2
NS = 16
NW = NC * NS
CHUNK_ROWS = N_PAD // (NW * 128)
CHUNK = CHUNK_ROWS * 128
GROUP = 5
ACC = 640


def _mlp_body(x_ref, w1_ref, b1_ref, w2r_ref, b2_ref, o_ref):
    x = x_ref[...]
    h = lax.dot_general(x, w1_ref[...], (((1,), (0,)), ((), ())),
                        preferred_element_type=jnp.float32)
    h = h + b1_ref[...]
    h = h * (1.0 / (1.0 + jnp.exp(-h)))
    c = lax.dot_general(w2r_ref[...], h, (((1,), (1,)), ((), ())),
                        preferred_element_type=jnp.float32)
    o_ref[...] = (c + b2_ref[0, 0]).reshape(ROWS)


def _mlp_contrib(x, w1, b1r, w2r, b2r):
    return pl.pallas_call(
        _mlp_body,
        grid=(GRID,),
        in_specs=[
            pl.BlockSpec((ROWS, D_FEAT), lambda i: (i, 0)),
            pl.BlockSpec((D_FEAT, HIDDEN), lambda i: (0, 0)),
            pl.BlockSpec((1, HIDDEN), lambda i: (0, 0)),
            pl.BlockSpec((1, HIDDEN), lambda i: (0, 0)),
            pl.BlockSpec((1, 1), lambda i: (0, 0)),
        ],
        out_specs=pl.BlockSpec((ROWS,), lambda i: (i,)),
        out_shape=jax.ShapeDtypeStruct((N_PAD,), jnp.float32),
    )(x, w1, b1r, w2r, b2r)


def _seg_body(contrib_hbm, batch_hbm, out_hbm,
              vals_v, idx_v, ones_v, zeros_v, ssum, scnt, sem):
    cid = lax.axis_index("c")
    sid = lax.axis_index("s")
    wid = sid * NC + cid

    one16 = jnp.ones((16,), jnp.float32)
    for k in range(8):
        ones_v[pl.ds(k * 16, 16)] = one16

    @pl.when(sid == 0)
    def _init():
        zero16 = jnp.zeros((16,), jnp.float32)
        for k in range(ACC // 16):
            zeros_v[pl.ds(k * 16, 16)] = zero16
        pltpu.sync_copy(zeros_v, ssum)
        pltpu.sync_copy(zeros_v, scnt)

    pltpu.sync_copy(contrib_hbm.at[pl.ds(wid * CHUNK, CHUNK)], vals_v)
    pltpu.sync_copy(batch_hbm.at[wid], idx_v)
    plsc.subcore_barrier()

    def body(g, carry):
        handles = []
        for u in range(GROUP):
            j = g * GROUP + u
            handles.append(pltpu.async_copy(
                vals_v.at[pl.ds(j * 128, 128)], ssum.at[idx_v.at[j]],
                sem, add=True))
            handles.append(pltpu.async_copy(
                ones_v, scnt.at[idx_v.at[j]], sem, add=True))
        for h in handles:
            h.wait()
        return carry

    lax.fori_loop(0, CHUNK_ROWS // GROUP, body, 0)
    plsc.subcore_barrier()

    @pl.when(sid == 0)
    def _publish():
        pltpu.sync_copy(ssum, out_hbm.at[cid, 0])
        pltpu.sync_copy(scnt, out_hbm.at[cid, 1])


@functools.cache
def _seg_partials():
    return pl.kernel(
        _seg_body,
        out_type=jax.ShapeDtypeStruct((NC, 2, ACC), jnp.float32),
        mesh=plsc.VectorSubcoreMesh(core_axis_name="c", subcore_axis_name="s",
                                    num_cores=NC, num_subcores=NS),
        scratch_types=[
            pltpu.VMEM((CHUNK,), jnp.float32),
            pltpu.VMEM((CHUNK_ROWS, 128), jnp.int32),
            pltpu.VMEM((128,), jnp.float32),
            pltpu.VMEM((ACC,), jnp.float32),
            pltpu.VMEM_SHARED((ACC,), jnp.float32),
            pltpu.VMEM_SHARED((ACC,), jnp.float32),
            pltpu.SemaphoreType.DMA,
        ],
    )


def _combine_body(p_ref, o_ref):
    p = p_ref[...]
    sums = p[0, 0, :NUM_SEGMENTS] + p[1, 0, :NUM_SEGMENTS]
    cnts = p[0, 1, :NUM_SEGMENTS] + p[1, 1, :NUM_SEGMENTS]
    o_ref[...] = (sums / jnp.maximum(cnts, 1.0)).reshape(1, NUM_SEGMENTS)


def _combine(partials):
    return pl.pallas_call(
        _combine_body,
        out_shape=jax.ShapeDtypeStruct((1, NUM_SEGMENTS), jnp.float32),
    )(partials)


def kernel(node_feats, batch, W1, b1, W2, b2):
    b1r = b1.reshape(1, HIDDEN)
    w2r = W2.reshape(1, HIDDEN)
    b2r = b2.reshape(1, 1)
    contrib = _mlp_contrib(node_feats, W1, b1r, w2r, b2r)

    batch_i = jnp.pad(batch.astype(jnp.int32), (0, N_PAD - N_NODES),
                      constant_values=NUM_SEGMENTS)
    batch_i = batch_i.reshape(NW, CHUNK_ROWS, 128)

    partials = _seg_partials()(contrib, batch_i)
    return _combine(partials).reshape(NUM_SEGMENTS)


# speedup vs baseline: 5.1853x; 1.1024x over previous
import functools

import jax
import jax.numpy as jnp
from jax import lax
from jax.experimental import pallas as pl
from jax.experimental.pallas import tpu as pltpu
from jax.experimental.pallas import tpu_sc as plsc

N_NODES = 100000
D_FEAT = 128
HIDDEN = 64
NUM_SEGMENTS = 512

ROWS = 10240
N_PAD = 102400
GRID = N_PAD // ROWS
NC = 2
NS = 16
NW = NC * NS
CHUNK_ROWS = N_PAD // (NW * 128)
CHUNK = CHUNK_ROWS * 128
GROUP = 5
ACC = 640


def _mlp_body(x_ref, w1_ref, b1_ref, w2r_ref, b2_ref, o_ref):
    x = x_ref[...]
    h = lax.dot_general(x, w1_ref[...], (((1,), (0,)), ((), ())),
                        preferred_element_type=jnp.float32)
    h = h + b1_ref[...]
    h = h * (1.0 / (1.0 + jnp.exp(-h)))
    c = lax.dot_general(w2r_ref[...], h, (((1,), (1,)), ((), ())),
                        preferred_element_type=jnp.float32)
    o_ref[...] = (c + b2_ref[0, 0]).reshape(ROWS)


def _mlp_contrib(x, w1, b1r, w2r, b2r):
    return pl.pallas_call(
        _mlp_body,
        grid=(GRID,),
        in_specs=[
            pl.BlockSpec((ROWS, D_FEAT), lambda i: (i, 0)),
            pl.BlockSpec((D_FEAT, HIDDEN), lambda i: (0, 0)),
            pl.BlockSpec((1, HIDDEN), lambda i: (0, 0)),
            pl.BlockSpec((1, HIDDEN), lambda i: (0, 0)),
            pl.BlockSpec((1, 1), lambda i: (0, 0)),
        ],
        out_specs=pl.BlockSpec((ROWS,), lambda i: (i,)),
        out_shape=jax.ShapeDtypeStruct((N_PAD,), jnp.float32),
    )(x, w1, b1r, w2r, b2r)


def _seg_body(contrib_hbm, batch_hbm, out_hbm,
              vals_v, idx_v, ones_v, zeros_v, ssum, scnt, sem):
    cid = lax.axis_index("c")
    sid = lax.axis_index("s")
    wid = sid * NC + cid

    one16 = jnp.ones((16,), jnp.float32)
    for k in range(8):
        ones_v[pl.ds(k * 16, 16)] = one16

    @pl.when(sid == 0)
    def _init():
        zero16 = jnp.zeros((16,), jnp.float32)
        for k in range(ACC // 16):
            zeros_v[pl.ds(k * 16, 16)] = zero16
        pltpu.sync_copy(zeros_v, ssum)
        pltpu.sync_copy(zeros_v, scnt)

    pltpu.sync_copy(contrib_hbm.at[pl.ds(wid * CHUNK, CHUNK)], vals_v)
    pltpu.sync_copy(batch_hbm.at[wid], idx_v)
    plsc.subcore_barrier()

    def body(g, carry):
        handles = []
        for u in range(GROUP):
            j = g * GROUP + u
            handles.append(pltpu.async_copy(
                vals_v.at[pl.ds(j * 128, 128)], ssum.at[idx_v.at[j]],
                sem, add=True))
            handles.append(pltpu.async_copy(
                ones_v, scnt.at[idx_v.at[j]], sem, add=True))
        for h in handles:
            h.wait()
        return carry

    lax.fori_loop(0, CHUNK_ROWS // GROUP, body, 0)
    plsc.subcore_barrier()

    @pl.when(sid == 0)
    def _publish():
        pltpu.sync_copy(ssum, out_hbm.at[cid, 0])
        pltpu.sync_copy(scnt, out_hbm.at[cid, 1])


@functools.cache
def _seg_partials():
    return pl.kernel(
        _seg_body,
        out_type=jax.ShapeDtypeStruct((NC, 2, ACC), jnp.float32),
        mesh=plsc.VectorSubcoreMesh(core_axis_name="c", subcore_axis_name="s",
                                    num_cores=NC, num_subcores=NS),
        scratch_types=[
            pltpu.VMEM((CHUNK,), jnp.float32),
            pltpu.VMEM((CHUNK_ROWS, 128), jnp.int32),
            pltpu.VMEM((128,), jnp.float32),
            pltpu.VMEM((ACC,), jnp.float32),
            pltpu.VMEM_SHARED((ACC,), jnp.float32),
            pltpu.VMEM_SHARED((ACC,), jnp.float32),
            pltpu.SemaphoreType.DMA,
        ],
    )


def _combine_body(p_ref, o_ref):
    p = p_ref[...]
    sums = p[0, 0, :NUM_SEGMENTS] + p[1, 0, :NUM_SEGMENTS]
    cnts = p[0, 1, :NUM_SEGMENTS] + p[1, 1, :NUM_SEGMENTS]
    o_ref[...] = (sums / jnp.maximum(cnts, 1.0)).reshape(1, NUM_SEGMENTS)


def _combine(partials):
    return pl.pallas_call(
        _combine_body,
        out_shape=jax.ShapeDtypeStruct((1, NUM_SEGMENTS), jnp.float32),
    )(partials)


def kernel(node_feats, batch, W1, b1, W2, b2):
    b1r = b1.reshape(1, HIDDEN)
    w2r = W2.reshape(1, HIDDEN)
    b2r = b2.reshape(1, 1)
    contrib = _mlp_contrib(node_feats, W1, b1r, w2r, b2r)

    batch_i = jnp.pad(batch.astype(jnp.int32), (0, N_PAD - N_NODES),
                      constant_values=NUM_SEGMENTS)
    batch_i = batch_i.reshape(NW, CHUNK_ROWS, 128)

    partials = _seg_partials()(contrib, batch_i)
    return _combine(partials).reshape(NUM_SEGMENTS)


# speedup vs baseline: 5.3612x; 1.0339x over previous
import functools

import jax
import jax.numpy as jnp
from jax import lax
from jax.experimental import pallas as pl
from jax.experimental.pallas import tpu as pltpu
from jax.experimental.pallas import tpu_sc as plsc

N_NODES = 100000
D_FEAT = 128
HIDDEN = 64
NUM_SEGMENTS = 512

ROWS = 20480
N_PAD = 102400
GRID = N_PAD // ROWS
NC = 2
NS = 16
NW = NC * NS
CHUNK_ROWS = N_PAD // (NW * 128)
CHUNK = CHUNK_ROWS * 128
GROUP = 5
ACC = 640


def _mlp_body(x_ref, w1_ref, b1_ref, w2r_ref, b2_ref, o_ref):
    x = x_ref[...]
    h = lax.dot_general(x, w1_ref[...], (((1,), (0,)), ((), ())),
                        preferred_element_type=jnp.float32)
    h = h + b1_ref[...]
    h = h * (1.0 / (1.0 + jnp.exp(-h)))
    c = lax.dot_general(w2r_ref[...], h, (((1,), (1,)), ((), ())),
                        preferred_element_type=jnp.float32)
    o_ref[...] = (c + b2_ref[0, 0]).reshape(ROWS)


def _mlp_contrib(x, w1, b1r, w2r, b2r):
    return pl.pallas_call(
        _mlp_body,
        grid=(GRID,),
        in_specs=[
            pl.BlockSpec((ROWS, D_FEAT), lambda i: (i, 0)),
            pl.BlockSpec((D_FEAT, HIDDEN), lambda i: (0, 0)),
            pl.BlockSpec((1, HIDDEN), lambda i: (0, 0)),
            pl.BlockSpec((1, HIDDEN), lambda i: (0, 0)),
            pl.BlockSpec((1, 1), lambda i: (0, 0)),
        ],
        out_specs=pl.BlockSpec((ROWS,), lambda i: (i,)),
        out_shape=jax.ShapeDtypeStruct((N_PAD,), jnp.float32),
    )(x, w1, b1r, w2r, b2r)


def _seg_body(contrib_hbm, batch_hbm, out_hbm,
              vals_v, idx_v, ones_v, zeros_v, ssum, scnt, sem):
    cid = lax.axis_index("c")
    sid = lax.axis_index("s")
    wid = sid * NC + cid

    one16 = jnp.ones((16,), jnp.float32)
    for k in range(8):
        ones_v[pl.ds(k * 16, 16)] = one16

    @pl.when(sid == 0)
    def _init():
        zero16 = jnp.zeros((16,), jnp.float32)
        for k in range(ACC // 16):
            zeros_v[pl.ds(k * 16, 16)] = zero16
        pltpu.sync_copy(zeros_v, ssum)
        pltpu.sync_copy(zeros_v, scnt)

    pltpu.sync_copy(contrib_hbm.at[pl.ds(wid * CHUNK, CHUNK)], vals_v)
    pltpu.sync_copy(batch_hbm.at[wid], idx_v)
    plsc.subcore_barrier()

    def body(g, carry):
        handles = []
        for u in range(GROUP):
            j = g * GROUP + u
            handles.append(pltpu.async_copy(
                vals_v.at[pl.ds(j * 128, 128)], ssum.at[idx_v.at[j]],
                sem, add=True))
            handles.append(pltpu.async_copy(
                ones_v, scnt.at[idx_v.at[j]], sem, add=True))
        for h in handles:
            h.wait()
        return carry

    lax.fori_loop(0, CHUNK_ROWS // GROUP, body, 0)
    plsc.subcore_barrier()

    @pl.when(sid == 0)
    def _publish():
        pltpu.sync_copy(ssum, out_hbm.at[cid, 0])
        pltpu.sync_copy(scnt, out_hbm.at[cid, 1])


@functools.cache
def _seg_partials():
    return pl.kernel(
        _seg_body,
        out_type=jax.ShapeDtypeStruct((NC, 2, ACC), jnp.float32),
        mesh=plsc.VectorSubcoreMesh(core_axis_name="c", subcore_axis_name="s",
                                    num_cores=NC, num_subcores=NS),
        scratch_types=[
            pltpu.VMEM((CHUNK,), jnp.float32),
            pltpu.VMEM((CHUNK_ROWS, 128), jnp.int32),
            pltpu.VMEM((128,), jnp.float32),
            pltpu.VMEM((ACC,), jnp.float32),
            pltpu.VMEM_SHARED((ACC,), jnp.float32),
            pltpu.VMEM_SHARED((ACC,), jnp.float32),
            pltpu.SemaphoreType.DMA,
        ],
    )


def _combine_body(p_ref, o_ref):
    p = p_ref[...]
    sums = p[0, 0, :NUM_SEGMENTS] + p[1, 0, :NUM_SEGMENTS]
    cnts = p[0, 1, :NUM_SEGMENTS] + p[1, 1, :NUM_SEGMENTS]
    o_ref[...] = (sums / jnp.maximum(cnts, 1.0)).reshape(1, NUM_SEGMENTS)


def _combine(partials):
    return pl.pallas_call(
        _combine_body,
        out_shape=jax.ShapeDtypeStruct((1, NUM_SEGMENTS), jnp.float32),
    )(partials)


def kernel(node_feats, batch, W1, b1, W2, b2):
    b1r = b1.reshape(1, HIDDEN)
    w2r = W2.reshape(1, HIDDEN)
    b2r = b2.reshape(1, 1)
    contrib = _mlp_contrib(node_feats, W1, b1r, w2r, b2r)

    batch_i = jnp.pad(batch.astype(jnp.int32), (0, N_PAD - N_NODES),
                      constant_values=NUM_SEGMENTS)
    batch_i = batch_i.reshape(NW, CHUNK_ROWS, 128)

    partials = _seg_partials()(contrib, batch_i)
    return _combine(partials).reshape(NUM_SEGMENTS)


# speedup vs baseline: 5.3825x; 1.0040x over previous
import functools

import jax
import jax.numpy as jnp
from jax import lax
from jax.experimental import pallas as pl
from jax.experimental.pallas import tpu as pltpu
from jax.experimental.pallas import tpu_sc as plsc

N_NODES = 100000
D_FEAT = 128
HIDDEN = 64
NUM_SEGMENTS = 512

ROWS = 25600
N_PAD = 102400
GRID = N_PAD // ROWS
NC = 2
NS = 16
NW = NC * NS
CHUNK_ROWS = N_PAD // (NW * 128)
CHUNK = CHUNK_ROWS * 128
GROUP = 5
ACC = 640


def _mlp_body(x_ref, w1_ref, b1_ref, w2r_ref, b2_ref, o_ref):
    x = x_ref[...]
    h = lax.dot_general(x, w1_ref[...], (((1,), (0,)), ((), ())),
                        preferred_element_type=jnp.float32)
    h = h + b1_ref[...]
    h = h * (1.0 / (1.0 + jnp.exp(-h)))
    c = lax.dot_general(w2r_ref[...], h, (((1,), (1,)), ((), ())),
                        preferred_element_type=jnp.float32)
    o_ref[...] = (c + b2_ref[0, 0]).reshape(ROWS)


def _mlp_contrib(x, w1, b1r, w2r, b2r):
    return pl.pallas_call(
        _mlp_body,
        grid=(GRID,),
        in_specs=[
            pl.BlockSpec((ROWS, D_FEAT), lambda i: (i, 0)),
            pl.BlockSpec((D_FEAT, HIDDEN), lambda i: (0, 0)),
            pl.BlockSpec((1, HIDDEN), lambda i: (0, 0)),
            pl.BlockSpec((1, HIDDEN), lambda i: (0, 0)),
            pl.BlockSpec((1, 1), lambda i: (0, 0)),
        ],
        out_specs=pl.BlockSpec((ROWS,), lambda i: (i,)),
        out_shape=jax.ShapeDtypeStruct((N_PAD,), jnp.float32),
    )(x, w1, b1r, w2r, b2r)


def _seg_body(contrib_hbm, batch_hbm, out_hbm,
              vals_v, idx_v, ones_v, zeros_v, ssum, scnt, sem):
    cid = lax.axis_index("c")
    sid = lax.axis_index("s")
    wid = sid * NC + cid

    one16 = jnp.ones((16,), jnp.float32)
    for k in range(8):
        ones_v[pl.ds(k * 16, 16)] = one16

    @pl.when(sid == 0)
    def _init():
        zero16 = jnp.zeros((16,), jnp.float32)
        for k in range(ACC // 16):
            zeros_v[pl.ds(k * 16, 16)] = zero16
        pltpu.sync_copy(zeros_v, ssum)
        pltpu.sync_copy(zeros_v, scnt)

    pltpu.sync_copy(contrib_hbm.at[pl.ds(wid * CHUNK, CHUNK)], vals_v)
    pltpu.sync_copy(batch_hbm.at[wid], idx_v)
    plsc.subcore_barrier()

    def body(g, carry):
        handles = []
        for u in range(GROUP):
            j = g * GROUP + u
            handles.append(pltpu.async_copy(
                vals_v.at[pl.ds(j * 128, 128)], ssum.at[idx_v.at[j]],
                sem, add=True))
            handles.append(pltpu.async_copy(
                ones_v, scnt.at[idx_v.at[j]], sem, add=True))
        for h in handles:
            h.wait()
        return carry

    lax.fori_loop(0, CHUNK_ROWS // GROUP, body, 0)
    plsc.subcore_barrier()

    @pl.when(sid == 0)
    def _publish():
        pltpu.sync_copy(ssum, out_hbm.at[cid, 0])
        pltpu.sync_copy(scnt, out_hbm.at[cid, 1])


@functools.cache
def _seg_partials():
    return pl.kernel(
        _seg_body,
        out_type=jax.ShapeDtypeStruct((NC, 2, ACC), jnp.float32),
        mesh=plsc.VectorSubcoreMesh(core_axis_name="c", subcore_axis_name="s",
                                    num_cores=NC, num_subcores=NS),
        scratch_types=[
            pltpu.VMEM((CHUNK,), jnp.float32),
            pltpu.VMEM((CHUNK_ROWS, 128), jnp.int32),
            pltpu.VMEM((128,), jnp.float32),
            pltpu.VMEM((ACC,), jnp.float32),
            pltpu.VMEM_SHARED((ACC,), jnp.float32),
            pltpu.VMEM_SHARED((ACC,), jnp.float32),
            pltpu.SemaphoreType.DMA,
        ],
    )


def _combine_body(p_ref, o_ref):
    p = p_ref[...]
    sums = p[0, 0, :NUM_SEGMENTS] + p[1, 0, :NUM_SEGMENTS]
    cnts = p[0, 1, :NUM_SEGMENTS] + p[1, 1, :NUM_SEGMENTS]
    o_ref[...] = (sums / jnp.maximum(cnts, 1.0)).reshape(1, NUM_SEGMENTS)


def _combine(partials):
    return pl.pallas_call(
        _combine_body,
        out_shape=jax.ShapeDtypeStruct((1, NUM_SEGMENTS), jnp.float32),
    )(partials)


def kernel(node_feats, batch, W1, b1, W2, b2):
    b1r = b1.reshape(1, HIDDEN)
    w2r = W2.reshape(1, HIDDEN)
    b2r = b2.reshape(1, 1)
    contrib = _mlp_contrib(node_feats, W1, b1r, w2r, b2r)

    batch_i = jnp.pad(batch.astype(jnp.int32), (0, N_PAD - N_NODES),
                      constant_values=NUM_SEGMENTS)
    batch_i = batch_i.reshape(NW, CHUNK_ROWS, 128)

    partials = _seg_partials()(contrib, batch_i)
    return _combine(partials).reshape(NUM_SEGMENTS)
